# Initial kernel scaffold; baseline (speedup 1.0000x reference)
import jax
import jax.numpy as jnp
from jax.experimental import pallas as pl


def kernel(x, edge_index, W1, att_src1, att_dst1, W_res1, b1, W2, att_src2, att_dst2, W_res2, b2):
    raise NotImplementedError("write your pallas kernel here")


# speedup vs baseline: 14.4516x; 14.4516x over previous
import functools

import jax
import jax.numpy as jnp
from jax import lax
from jax.experimental import pallas as pl
from jax.experimental.pallas import tpu as pltpu
from jax.experimental.pallas import tpu_sc as plsc

N = 10000
E = 320000
NC, NS, L = 2, 16, 16
NP = 10240
RPT = NP // NS
ZR = 128
B = 128
G = E // B
CW = 128


def _bcast_lane(v16, r2):
    return lax.gather(
        v16, jnp.full((L, 1), r2, jnp.int32),
        lax.GatherDimensionNumbers(
            offset_dims=(), collapsed_slice_dims=(0,), start_index_map=(0,)),
        (1,),
        mode=lax.GatherScatterMode.PROMISE_IN_BOUNDS)


def _sc_edge(split_edges):
    f32 = jnp.float32
    mesh = plsc.VectorSubcoreMesh(core_axis_name="c", subcore_axis_name="s")
    GC = G // NC
    CH_I = (G + NS - 1) // NS if not split_edges else (GC + NS - 1) // NS

    @functools.partial(
        pl.kernel,
        out_type=(jax.ShapeDtypeStruct((NC * NP, CW), f32),
                  jax.ShapeDtypeStruct((NC * NP,), f32)),
        mesh=mesh,
        scratch_types=[
            pltpu.VMEM_SHARED((NP, CW), f32),
            pltpu.VMEM_SHARED((NP,), f32),
            pltpu.VMEM((B,), jnp.int32),
            pltpu.VMEM((B,), jnp.int32),
            pltpu.VMEM((B,), jnp.int32),
            pltpu.VMEM((B,), f32),
            pltpu.VMEM((B,), f32),
            pltpu.VMEM((B,), f32),
            pltpu.VMEM((B, CW), f32),
            pltpu.VMEM((ZR, CW), f32),
            pltpu.VMEM((RPT,), f32),
            pltpu.SemaphoreType.DMA,
        ],
    )
    def k(h2, asrc, adst, src, dst, agg_o, den_o,
          accum_sp, den_sp, src_v, dst_v, sidx_v, asg_v, adg_v, w_v,
          rows_v, zbuf, zden, sem):
        c = lax.axis_index("c")
        sid = lax.axis_index("s")

        def zrow(r, carry):
            for j in range(CW // L):
                zbuf[r, pl.ds(j * L, L)] = jnp.zeros((L,), f32)
            return carry
        lax.fori_loop(0, ZR, zrow, 0)
        for j in range(RPT // L):
            zden[pl.ds(j * L, L)] = jnp.zeros((L,), f32)
        for kk in range(RPT // ZR):
            pltpu.sync_copy(zbuf, accum_sp.at[pl.ds(sid * RPT + kk * ZR, ZR)])
        pltpu.sync_copy(zden, den_sp.at[pl.ds(sid * RPT, RPT)])
        plsc.subcore_barrier()

        if split_edges:
            g0 = c * GC
            glim = g0 + GC
            coff = 0
        else:
            g0 = 0
            glim = G
            coff = c * N

        def chunk(i, carry):
            g = g0 + sid + NS * i

            @pl.when(g < glim)
            def _():
                base = g * B
                pltpu.sync_copy(src.at[pl.ds(base, B)], src_v)
                pltpu.sync_copy(dst.at[pl.ds(base, B)], dst_v)
                pltpu.async_copy(asrc.at[src_v], asg_v, sem).wait()
                pltpu.async_copy(adst.at[dst_v], adg_v, sem).wait()
                for j in range(B // L):
                    sl = pl.ds(j * L, L)
                    a = asg_v[sl] + adg_v[sl]
                    e = jnp.where(a >= 0.0, a, 0.2 * a)
                    w_v[sl] = jnp.exp(e)
                    sidx_v[sl] = src_v[sl] + coff
                pltpu.async_copy(h2.at[sidx_v], rows_v, sem).wait()

                def srow16(eb, carry2):
                    w16 = w_v[pl.ds(eb * L, L)]
                    for r2 in range(L):
                        wb = _bcast_lane(w16, r2)
                        r = eb * L + r2
                        for j2 in range(CW // L):
                            sl2 = pl.ds(j2 * L, L)
                            rows_v[r, sl2] = rows_v[r, sl2] * wb
                    return carry2
                lax.fori_loop(0, B // L, srow16, 0)

                pltpu.sync_copy(rows_v, accum_sp.at[dst_v], add=True)
                pltpu.sync_copy(w_v, den_sp.at[dst_v], add=True)
            return carry
        lax.fori_loop(0, CH_I, chunk, 0)

        plsc.subcore_barrier()

        for kk in range(RPT // ZR):
            r0 = sid * RPT + kk * ZR
            pltpu.sync_copy(accum_sp.at[pl.ds(r0, ZR)], zbuf)
            pltpu.sync_copy(zbuf, agg_o.at[pl.ds(c * NP + r0, ZR)])
        pltpu.sync_copy(den_sp.at[pl.ds(sid * RPT, RPT)], zden)
        pltpu.sync_copy(zden, den_o.at[pl.ds(c * NP + sid * RPT, RPT)])

    return k


def _tc_pre(D, C):
    BN = 400
    f32 = jnp.float32

    def body(x_r, w_r, s_r, d_r, h_r, as_r, ad_r):
        h = jnp.dot(x_r[...], w_r[...], preferred_element_type=f32,
                    precision=lax.Precision.HIGHEST)
        h_r[...] = h
        as_r[...] = jnp.sum(h * s_r[...], axis=1, keepdims=True)
        ad_r[...] = jnp.sum(h * d_r[...], axis=1, keepdims=True)

    return pl.pallas_call(
        body,
        grid=(N // BN,),
        in_specs=[
            pl.BlockSpec((BN, D), lambda i: (i, 0)),
            pl.BlockSpec((D, C), lambda i: (0, 0)),
            pl.BlockSpec((1, C), lambda i: (0, 0)),
            pl.BlockSpec((1, C), lambda i: (0, 0)),
        ],
        out_specs=[
            pl.BlockSpec((BN, C), lambda i: (i, 0)),
            pl.BlockSpec((BN, 1), lambda i: (i, 0)),
            pl.BlockSpec((BN, 1), lambda i: (i, 0)),
        ],
        out_shape=[
            jax.ShapeDtypeStruct((N, C), f32),
            jax.ShapeDtypeStruct((N, 1), f32),
            jax.ShapeDtypeStruct((N, 1), f32),
        ],
    )


def _tc_combine(D, C, relu, merge):
    BN = 400
    f32 = jnp.float32

    def body(*refs):
        if merge:
            a0_r, a1_r, d0_r, d1_r, x_r, wr_r, b_r, o_r = refs
            agg = a0_r[...] + a1_r[...]
            den = d0_r[...] + d1_r[...]
        else:
            a0_r, d0_r, x_r, wr_r, b_r, o_r = refs
            agg = a0_r[...]
            den = d0_r[...]
        r = jnp.dot(x_r[...], wr_r[...], preferred_element_type=f32,
                    precision=lax.Precision.HIGHEST) + b_r[...]
        o = agg / (den + 1e-16) + r
        if relu:
            o = jnp.maximum(o, 0.0)
        o_r[...] = o

    agg_spec = pl.BlockSpec((BN, C), lambda i: (i, 0))
    den_spec = pl.BlockSpec((BN, 1), lambda i: (i, 0))
    in_specs = [agg_spec, agg_spec, den_spec, den_spec] if merge else \
               [agg_spec, den_spec]
    in_specs += [
        pl.BlockSpec((BN, D), lambda i: (i, 0)),
        pl.BlockSpec((D, C), lambda i: (0, 0)),
        pl.BlockSpec((1, C), lambda i: (0, 0)),
    ]
    return pl.pallas_call(
        body,
        grid=(N // BN,),
        in_specs=in_specs,
        out_specs=pl.BlockSpec((BN, C), lambda i: (i, 0)),
        out_shape=jax.ShapeDtypeStruct((N, C), f32),
    )


def _gat_layer(x, src, dst, W, att_s, att_d, W_res, b, relu):
    D = x.shape[1]
    C = W.shape[1]
    h, asrc, adst = _tc_pre(D, C)(x, W, att_s.reshape(1, C),
                                  att_d.reshape(1, C))
    if C == CW:
        aggf, denf = _sc_edge(True)(h, asrc.reshape(N), adst.reshape(N),
                                    src, dst)
        return _tc_combine(D, C, relu, True)(
            aggf[:N], aggf[NP:NP + N],
            denf[:N].reshape(N, 1), denf[NP:NP + N].reshape(N, 1),
            x, W_res, b.reshape(1, C))
    else:
        C2 = C // 2
        assert C2 == CW
        h2 = jnp.concatenate([h[:, :C2], h[:, C2:]], axis=0)
        aggf, denf = _sc_edge(False)(h2, asrc.reshape(N), adst.reshape(N),
                                     src, dst)
        agg = jnp.concatenate([aggf[:N], aggf[NP:NP + N]], axis=1)
        return _tc_combine(D, C, relu, False)(
            agg, denf[:N].reshape(N, 1), x, W_res, b.reshape(1, C))


def kernel(x, edge_index, W1, att_src1, att_dst1, W_res1, b1,
           W2, att_src2, att_dst2, W_res2, b2):
    src = edge_index[0].astype(jnp.int32)
    dst = edge_index[1].astype(jnp.int32)
    h_mid = _gat_layer(x, src, dst, W1, att_src1, att_dst1, W_res1, b1,
                       relu=True)
    return _gat_layer(h_mid, src, dst, W2, att_src2, att_dst2, W_res2, b2,
                      relu=False)


# speedup vs baseline: 24.1538x; 1.6714x over previous
import functools

import jax
import jax.numpy as jnp
from jax import lax
from jax.experimental import pallas as pl
from jax.experimental.pallas import tpu as pltpu
from jax.experimental.pallas import tpu_sc as plsc

N = 10000
E = 320000
NC, NS, L = 2, 16, 16
NP = 10240
RPT = NP // NS
ZR = 128
B = 128
G = E // B
CW = 128


def _bcast_lane(v16, r2):
    return lax.gather(
        v16, jnp.full((L, 1), r2, jnp.int32),
        lax.GatherDimensionNumbers(
            offset_dims=(), collapsed_slice_dims=(0,), start_index_map=(0,)),
        (1,),
        mode=lax.GatherScatterMode.PROMISE_IN_BOUNDS)


def _sc_edge(split_edges):
    f32 = jnp.float32
    mesh = plsc.VectorSubcoreMesh(core_axis_name="c", subcore_axis_name="s")
    GC = G // NC
    CH_I = (G + NS - 1) // NS if not split_edges else (GC + NS - 1) // NS
    NB = 2
    NIT = (CH_I + NB - 1) // NB

    @functools.partial(
        pl.kernel,
        out_type=(jax.ShapeDtypeStruct((NC * NP, CW), f32),
                  jax.ShapeDtypeStruct((NC * NP,), f32)),
        mesh=mesh,
        scratch_types=(
            [pltpu.VMEM_SHARED((NP, CW), f32),
             pltpu.VMEM_SHARED((NP,), f32)]
            + [pltpu.VMEM((B,), jnp.int32)] * NB
            + [pltpu.VMEM((B,), jnp.int32)] * NB
            + [pltpu.VMEM((B,), jnp.int32)] * NB
            + [pltpu.VMEM((B,), f32)] * NB
            + [pltpu.VMEM((B,), f32)] * NB
            + [pltpu.VMEM((B,), f32)] * NB
            + [pltpu.VMEM((B, CW), f32)] * NB
            + [pltpu.VMEM((RPT,), f32)]
            + [pltpu.SemaphoreType.DMA] * (3 * NB)
        ),
    )
    def k(h2, asrc, adst, src, dst, agg_o, den_o, accum_sp, den_sp, *rest):
        src_v = rest[0:NB]
        dst_v = rest[NB:2 * NB]
        sidx_v = rest[2 * NB:3 * NB]
        asg_v = rest[3 * NB:4 * NB]
        adg_v = rest[4 * NB:5 * NB]
        w_v = rest[5 * NB:6 * NB]
        rows_v = rest[6 * NB:7 * NB]
        zbuf = rows_v[0]
        zden = rest[7 * NB]
        semI = rest[7 * NB + 1:7 * NB + 1 + NB]
        semS = rest[7 * NB + 1 + NB:7 * NB + 1 + 2 * NB]
        semR = rest[7 * NB + 1 + 2 * NB:7 * NB + 1 + 3 * NB]
        c = lax.axis_index("c")
        sid = lax.axis_index("s")

        def zrow(r, carry):
            for j in range(CW // L):
                zbuf[r, pl.ds(j * L, L)] = jnp.zeros((L,), f32)
            return carry
        lax.fori_loop(0, ZR, zrow, 0)
        for j in range(RPT // L):
            zden[pl.ds(j * L, L)] = jnp.zeros((L,), f32)
        for kk in range(RPT // ZR):
            pltpu.sync_copy(zbuf, accum_sp.at[pl.ds(sid * RPT + kk * ZR, ZR)])
        pltpu.sync_copy(zden, den_sp.at[pl.ds(sid * RPT, RPT)])
        plsc.subcore_barrier()

        if split_edges:
            g0 = c * GC
            glim = g0 + GC
            coff = 0
        else:
            g0 = 0
            glim = G
            coff = c * N

        def chunk4(it, carry):
            gs = [g0 + sid + NS * (NB * it + b) for b in range(NB)]
            descs = {}

            for b in range(NB):
                @pl.when(gs[b] < glim)
                def _(b=b):
                    descs[("si", b)] = pltpu.async_copy(
                        src.at[gs[b]], src_v[b], semI[b])
                    descs[("di", b)] = pltpu.async_copy(
                        dst.at[gs[b]], dst_v[b], semI[b])

            for b in range(NB):
                @pl.when(gs[b] < glim)
                def _(b=b):
                    descs[("si", b)].wait()
                    descs[("di", b)].wait()
                    descs[("as", b)] = pltpu.async_copy(
                        asrc.at[src_v[b]], asg_v[b], semS[b])
                    descs[("ad", b)] = pltpu.async_copy(
                        adst.at[dst_v[b]], adg_v[b], semS[b])
                    if split_edges:
                        ridx = src_v[b]
                    else:
                        for j in range(B // L):
                            sl = pl.ds(j * L, L)
                            sidx_v[b][sl] = src_v[b][sl] + coff
                        ridx = sidx_v[b]
                    descs[("r", b)] = pltpu.async_copy(
                        h2.at[ridx], rows_v[b], semR[b])

            for b in range(NB):
                @pl.when(gs[b] < glim)
                def _(b=b):
                    descs[("as", b)].wait()
                    descs[("ad", b)].wait()
                    for j in range(B // L):
                        sl = pl.ds(j * L, L)
                        a = asg_v[b][sl] + adg_v[b][sl]
                        e = jnp.where(a >= 0.0, a, 0.2 * a)
                        w_v[b][sl] = jnp.exp(e)
                    descs[("r", b)].wait()

                    def srow16(eb, carry2):
                        w16 = w_v[b][pl.ds(eb * L, L)]
                        for r2 in range(L):
                            wb = _bcast_lane(w16, r2)
                            r = eb * L + r2
                            for j2 in range(CW // L):
                                sl2 = pl.ds(j2 * L, L)
                                rows_v[b][r, sl2] = rows_v[b][r, sl2] * wb
                        return carry2
                    lax.fori_loop(0, B // L, srow16, 0)

                    pltpu.sync_copy(rows_v[b], accum_sp.at[dst_v[b]],
                                    add=True)
                    pltpu.sync_copy(w_v[b], den_sp.at[dst_v[b]], add=True)
            return carry
        lax.fori_loop(0, NIT, chunk4, 0)

        plsc.subcore_barrier()

        for kk in range(RPT // ZR):
            r0 = sid * RPT + kk * ZR
            pltpu.sync_copy(accum_sp.at[pl.ds(r0, ZR)], zbuf)
            pltpu.sync_copy(zbuf, agg_o.at[pl.ds(c * NP + r0, ZR)])
        pltpu.sync_copy(den_sp.at[pl.ds(sid * RPT, RPT)], zden)
        pltpu.sync_copy(zden, den_o.at[pl.ds(c * NP + sid * RPT, RPT)])

    return k


def _tc_pre(D, C):
    BN = 400
    f32 = jnp.float32

    def body(x_r, w_r, s_r, d_r, h_r, as_r, ad_r):
        h = jnp.dot(x_r[...], w_r[...], preferred_element_type=f32,
                    precision=lax.Precision.HIGHEST)
        h_r[...] = h
        as_r[...] = jnp.sum(h * s_r[...], axis=1, keepdims=True)
        ad_r[...] = jnp.sum(h * d_r[...], axis=1, keepdims=True)

    return pl.pallas_call(
        body,
        grid=(N // BN,),
        in_specs=[
            pl.BlockSpec((BN, D), lambda i: (i, 0)),
            pl.BlockSpec((D, C), lambda i: (0, 0)),
            pl.BlockSpec((1, C), lambda i: (0, 0)),
            pl.BlockSpec((1, C), lambda i: (0, 0)),
        ],
        out_specs=[
            pl.BlockSpec((BN, C), lambda i: (i, 0)),
            pl.BlockSpec((BN, 1), lambda i: (i, 0)),
            pl.BlockSpec((BN, 1), lambda i: (i, 0)),
        ],
        out_shape=[
            jax.ShapeDtypeStruct((N, C), f32),
            jax.ShapeDtypeStruct((N, 1), f32),
            jax.ShapeDtypeStruct((N, 1), f32),
        ],
    )


def _tc_combine(D, C, relu, merge):
    BN = 400
    f32 = jnp.float32

    def body(*refs):
        if merge:
            a0_r, a1_r, d0_r, d1_r, x_r, wr_r, b_r, o_r = refs
            agg = a0_r[...] + a1_r[...]
            den = d0_r[...] + d1_r[...]
        else:
            a0_r, d0_r, x_r, wr_r, b_r, o_r = refs
            agg = a0_r[...]
            den = d0_r[...]
        r = jnp.dot(x_r[...], wr_r[...], preferred_element_type=f32,
                    precision=lax.Precision.HIGHEST) + b_r[...]
        o = agg / (den + 1e-16) + r
        if relu:
            o = jnp.maximum(o, 0.0)
        o_r[...] = o

    agg_spec = pl.BlockSpec((BN, C), lambda i: (i, 0))
    den_spec = pl.BlockSpec((BN, 1), lambda i: (i, 0))
    in_specs = [agg_spec, agg_spec, den_spec, den_spec] if merge else \
               [agg_spec, den_spec]
    in_specs += [
        pl.BlockSpec((BN, D), lambda i: (i, 0)),
        pl.BlockSpec((D, C), lambda i: (0, 0)),
        pl.BlockSpec((1, C), lambda i: (0, 0)),
    ]
    return pl.pallas_call(
        body,
        grid=(N // BN,),
        in_specs=in_specs,
        out_specs=pl.BlockSpec((BN, C), lambda i: (i, 0)),
        out_shape=jax.ShapeDtypeStruct((N, C), f32),
    )


def _gat_layer(x, src, dst, W, att_s, att_d, W_res, b, relu):
    D = x.shape[1]
    C = W.shape[1]
    h, asrc, adst = _tc_pre(D, C)(x, W, att_s.reshape(1, C),
                                  att_d.reshape(1, C))
    if C == CW:
        aggf, denf = _sc_edge(True)(h, asrc.reshape(N), adst.reshape(N),
                                    src.reshape(G, B), dst.reshape(G, B))
        return _tc_combine(D, C, relu, True)(
            aggf[:N], aggf[NP:NP + N],
            denf[:N].reshape(N, 1), denf[NP:NP + N].reshape(N, 1),
            x, W_res, b.reshape(1, C))
    else:
        C2 = C // 2
        assert C2 == CW
        h2 = jnp.concatenate([h[:, :C2], h[:, C2:]], axis=0)
        aggf, denf = _sc_edge(False)(h2, asrc.reshape(N), adst.reshape(N),
                                     src.reshape(G, B), dst.reshape(G, B))
        agg = jnp.concatenate([aggf[:N], aggf[NP:NP + N]], axis=1)
        return _tc_combine(D, C, relu, False)(
            agg, denf[:N].reshape(N, 1), x, W_res, b.reshape(1, C))


def kernel(x, edge_index, W1, att_src1, att_dst1, W_res1, b1,
           W2, att_src2, att_dst2, W_res2, b2):
    src = edge_index[0].astype(jnp.int32)
    dst = edge_index[1].astype(jnp.int32)
    h_mid = _gat_layer(x, src, dst, W1, att_src1, att_dst1, W_res1, b1,
                       relu=True)
    return _gat_layer(h_mid, src, dst, W2, att_src2, att_dst2, W_res2, b2,
                      relu=False)


# speedup vs baseline: 35.5545x; 1.4720x over previous
import functools

import jax
import jax.numpy as jnp
from jax import lax
from jax.experimental import pallas as pl
from jax.experimental.pallas import tpu as pltpu
from jax.experimental.pallas import tpu_sc as plsc

N = 10000
E = 320000
NC, NS, L = 2, 16, 16
NP = 10240
RPT = NP // NS
ZR = 128
B = 128
G = E // B
CW = 128


def _bcast_lane(v16, r2):
    return lax.gather(
        v16, jnp.full((L, 1), r2, jnp.int32),
        lax.GatherDimensionNumbers(
            offset_dims=(), collapsed_slice_dims=(0,), start_index_map=(0,)),
        (1,),
        mode=lax.GatherScatterMode.PROMISE_IN_BOUNDS)


def _sc_edge(split_edges):
    f32 = jnp.float32
    mesh = plsc.VectorSubcoreMesh(core_axis_name="c", subcore_axis_name="s")
    GC = G // NC
    CH_I = (G + NS - 1) // NS if not split_edges else (GC + NS - 1) // NS
    NB = 2
    NIT = (CH_I + NB - 1) // NB

    @functools.partial(
        pl.kernel,
        out_type=(jax.ShapeDtypeStruct((NC * NP, CW), f32),
                  jax.ShapeDtypeStruct((NC * NP,), f32)),
        mesh=mesh,
        scratch_types=(
            [pltpu.VMEM_SHARED((NP, CW), f32),
             pltpu.VMEM_SHARED((NP,), f32)]
            + [pltpu.VMEM((B,), jnp.int32)] * NB
            + [pltpu.VMEM((B,), jnp.int32)] * NB
            + [pltpu.VMEM((B,), jnp.int32)] * NB
            + [pltpu.VMEM((B,), f32)] * NB
            + [pltpu.VMEM((B,), f32)] * NB
            + [pltpu.VMEM((B,), jnp.int32)] * NB
            + [pltpu.VMEM((B,), f32)] * NB
            + [pltpu.VMEM((B, CW), f32)] * NB
            + [pltpu.VMEM((RPT,), f32)]
            + [pltpu.SemaphoreType.DMA] * (3 * NB)
        ),
    )
    def k(h2, asrc, adst, src, dst, agg_o, den_o, accum_sp, den_sp, *rest):
        src_v = rest[0:NB]
        dst_v = rest[NB:2 * NB]
        sidx_v = rest[2 * NB:3 * NB]
        asg_v = rest[3 * NB:4 * NB]
        adg_v = rest[4 * NB:5 * NB]
        dst_c = rest[5 * NB:6 * NB]
        w_v = rest[6 * NB:7 * NB]
        rows_v = rest[7 * NB:8 * NB]
        zbuf = rows_v[0]
        zden = rest[8 * NB]
        semI = rest[8 * NB + 1:8 * NB + 1 + NB]
        semS = rest[8 * NB + 1 + NB:8 * NB + 1 + 2 * NB]
        semR = rest[8 * NB + 1 + 2 * NB:8 * NB + 1 + 3 * NB]
        c = lax.axis_index("c")
        sid = lax.axis_index("s")

        def zrow(r, carry):
            for j in range(CW // L):
                zbuf[r, pl.ds(j * L, L)] = jnp.zeros((L,), f32)
            return carry
        lax.fori_loop(0, ZR, zrow, 0)
        for j in range(RPT // L):
            zden[pl.ds(j * L, L)] = jnp.zeros((L,), f32)
        for kk in range(RPT // ZR):
            pltpu.sync_copy(zbuf, accum_sp.at[pl.ds(sid * RPT + kk * ZR, ZR)])
        pltpu.sync_copy(zden, den_sp.at[pl.ds(sid * RPT, RPT)])
        plsc.subcore_barrier()

        if split_edges:
            g0 = c * GC
            glim = g0 + GC
            coff = 0
        else:
            g0 = 0
            glim = G
            coff = c * N

        def fire_idx(g, b):
            @pl.when(g < glim)
            def _():
                pltpu.async_copy(src.at[g], src_v[b], semI[b])
                pltpu.async_copy(dst.at[g], dst_v[b], semI[b])

        def wait_idx_fire_gath(g, b):
            @pl.when(g < glim)
            def _():
                pltpu.make_async_copy(src.at[0], src_v[b], semI[b]).wait()
                pltpu.make_async_copy(dst.at[0], dst_v[b], semI[b]).wait()
                pltpu.async_copy(asrc.at[src_v[b]], asg_v[b], semS[b])
                pltpu.async_copy(adst.at[dst_v[b]], adg_v[b], semS[b])
                if split_edges:
                    ridx = src_v[b]
                else:
                    for j in range(B // L):
                        sl = pl.ds(j * L, L)
                        sidx_v[b][sl] = src_v[b][sl] + coff
                    ridx = sidx_v[b]
                pltpu.async_copy(h2.at[ridx], rows_v[b], semR[b])

        def wait_gath_weights(g, b):
            @pl.when(g < glim)
            def _():
                pltpu.make_async_copy(
                    asrc.at[pl.ds(0, B)], asg_v[b], semS[b]).wait()
                pltpu.make_async_copy(
                    adst.at[pl.ds(0, B)], adg_v[b], semS[b]).wait()
                for j in range(B // L):
                    sl = pl.ds(j * L, L)
                    a = asg_v[b][sl] + adg_v[b][sl]
                    e = jnp.where(a >= 0.0, a, 0.2 * a)
                    w_v[b][sl] = jnp.exp(e)
                    dst_c[b][sl] = dst_v[b][sl]
                pltpu.make_async_copy(
                    h2.at[pl.ds(0, B)], rows_v[b], semR[b]).wait()

        def scale_scatter(g, b):
            @pl.when(g < glim)
            def _():
                def srow16(eb, carry2):
                    w16 = w_v[b][pl.ds(eb * L, L)]
                    for r2 in range(L):
                        wb = _bcast_lane(w16, r2)
                        r = eb * L + r2
                        for j2 in range(CW // L):
                            sl2 = pl.ds(j2 * L, L)
                            rows_v[b][r, sl2] = rows_v[b][r, sl2] * wb
                    return carry2
                lax.fori_loop(0, B // L, srow16, 0)
                pltpu.sync_copy(rows_v[b], accum_sp.at[dst_c[b]], add=True)
                pltpu.sync_copy(w_v[b], den_sp.at[dst_c[b]], add=True)

        ga0 = g0 + sid
        fire_idx(ga0, 0)
        wait_idx_fire_gath(ga0, 0)
        fire_idx(ga0 + NS, 1)

        def body(it, carry):
            ga = g0 + sid + NS * 2 * it
            gb = ga + NS
            gc = ga + 2 * NS
            gd = ga + 3 * NS
            wait_idx_fire_gath(gb, 1)
            wait_gath_weights(ga, 0)
            fire_idx(gc, 0)
            scale_scatter(ga, 0)
            wait_idx_fire_gath(gc, 0)
            wait_gath_weights(gb, 1)
            fire_idx(gd, 1)
            scale_scatter(gb, 1)
            return carry
        lax.fori_loop(0, NIT, body, 0)

        plsc.subcore_barrier()

        for kk in range(RPT // ZR):
            r0 = sid * RPT + kk * ZR
            pltpu.sync_copy(accum_sp.at[pl.ds(r0, ZR)], zbuf)
            pltpu.sync_copy(zbuf, agg_o.at[pl.ds(c * NP + r0, ZR)])
        pltpu.sync_copy(den_sp.at[pl.ds(sid * RPT, RPT)], zden)
        pltpu.sync_copy(zden, den_o.at[pl.ds(c * NP + sid * RPT, RPT)])

    return k


def _tc_pre(D, C):
    BN = 400
    f32 = jnp.float32

    def body(x_r, w_r, s_r, d_r, h_r, as_r, ad_r):
        h = jnp.dot(x_r[...], w_r[...], preferred_element_type=f32,
                    precision=lax.Precision.HIGHEST)
        h_r[...] = h
        as_r[...] = jnp.sum(h * s_r[...], axis=1, keepdims=True)
        ad_r[...] = jnp.sum(h * d_r[...], axis=1, keepdims=True)

    return pl.pallas_call(
        body,
        grid=(N // BN,),
        in_specs=[
            pl.BlockSpec((BN, D), lambda i: (i, 0)),
            pl.BlockSpec((D, C), lambda i: (0, 0)),
            pl.BlockSpec((1, C), lambda i: (0, 0)),
            pl.BlockSpec((1, C), lambda i: (0, 0)),
        ],
        out_specs=[
            pl.BlockSpec((BN, C), lambda i: (i, 0)),
            pl.BlockSpec((BN, 1), lambda i: (i, 0)),
            pl.BlockSpec((BN, 1), lambda i: (i, 0)),
        ],
        out_shape=[
            jax.ShapeDtypeStruct((N, C), f32),
            jax.ShapeDtypeStruct((N, 1), f32),
            jax.ShapeDtypeStruct((N, 1), f32),
        ],
    )


def _tc_combine(D, C, relu, merge):
    BN = 400
    f32 = jnp.float32

    def body(*refs):
        if merge:
            a0_r, a1_r, d0_r, d1_r, x_r, wr_r, b_r, o_r = refs
            agg = a0_r[...] + a1_r[...]
            den = d0_r[...] + d1_r[...]
        else:
            a0_r, d0_r, x_r, wr_r, b_r, o_r = refs
            agg = a0_r[...]
            den = d0_r[...]
        r = jnp.dot(x_r[...], wr_r[...], preferred_element_type=f32,
                    precision=lax.Precision.HIGHEST) + b_r[...]
        o = agg / (den + 1e-16) + r
        if relu:
            o = jnp.maximum(o, 0.0)
        o_r[...] = o

    agg_spec = pl.BlockSpec((BN, C), lambda i: (i, 0))
    den_spec = pl.BlockSpec((BN, 1), lambda i: (i, 0))
    in_specs = [agg_spec, agg_spec, den_spec, den_spec] if merge else \
               [agg_spec, den_spec]
    in_specs += [
        pl.BlockSpec((BN, D), lambda i: (i, 0)),
        pl.BlockSpec((D, C), lambda i: (0, 0)),
        pl.BlockSpec((1, C), lambda i: (0, 0)),
    ]
    return pl.pallas_call(
        body,
        grid=(N // BN,),
        in_specs=in_specs,
        out_specs=pl.BlockSpec((BN, C), lambda i: (i, 0)),
        out_shape=jax.ShapeDtypeStruct((N, C), f32),
    )


def _gat_layer(x, src, dst, W, att_s, att_d, W_res, b, relu):
    D = x.shape[1]
    C = W.shape[1]
    h, asrc, adst = _tc_pre(D, C)(x, W, att_s.reshape(1, C),
                                  att_d.reshape(1, C))
    if C == CW:
        aggf, denf = _sc_edge(True)(h, asrc.reshape(N), adst.reshape(N),
                                    src.reshape(G, B), dst.reshape(G, B))
        return _tc_combine(D, C, relu, True)(
            aggf[:N], aggf[NP:NP + N],
            denf[:N].reshape(N, 1), denf[NP:NP + N].reshape(N, 1),
            x, W_res, b.reshape(1, C))
    else:
        C2 = C // 2
        assert C2 == CW
        h2 = jnp.concatenate([h[:, :C2], h[:, C2:]], axis=0)
        aggf, denf = _sc_edge(False)(h2, asrc.reshape(N), adst.reshape(N),
                                     src.reshape(G, B), dst.reshape(G, B))
        agg = jnp.concatenate([aggf[:N], aggf[NP:NP + N]], axis=1)
        return _tc_combine(D, C, relu, False)(
            agg, denf[:N].reshape(N, 1), x, W_res, b.reshape(1, C))


def kernel(x, edge_index, W1, att_src1, att_dst1, W_res1, b1,
           W2, att_src2, att_dst2, W_res2, b2):
    src = edge_index[0].astype(jnp.int32)
    dst = edge_index[1].astype(jnp.int32)
    h_mid = _gat_layer(x, src, dst, W1, att_src1, att_dst1, W_res1, b1,
                       relu=True)
    return _gat_layer(h_mid, src, dst, W2, att_src2, att_dst2, W_res2, b2,
                      relu=False)


# speedup vs baseline: 38.1656x; 1.0734x over previous
import functools

import jax
import jax.numpy as jnp
from jax import lax
from jax.experimental import pallas as pl
from jax.experimental.pallas import tpu as pltpu
from jax.experimental.pallas import tpu_sc as plsc

N = 10000
E = 320000
NC, NS, L = 2, 16, 16
NP = 10240
RPT = NP // NS
ZR = 128
B = 128
G = E // B
CW = 128


def _bcast_lane(v16, r2):
    return lax.gather(
        v16, jnp.full((L, 1), r2, jnp.int32),
        lax.GatherDimensionNumbers(
            offset_dims=(), collapsed_slice_dims=(0,), start_index_map=(0,)),
        (1,),
        mode=lax.GatherScatterMode.PROMISE_IN_BOUNDS)


def _sc_edge(split_edges):
    f32 = jnp.float32
    mesh = plsc.VectorSubcoreMesh(core_axis_name="c", subcore_axis_name="s")
    GC = G // NC
    CH_I = (G + NS - 1) // NS if not split_edges else (GC + NS - 1) // NS
    NB = 2
    NIT = (CH_I + NB - 1) // NB

    @functools.partial(
        pl.kernel,
        out_type=(jax.ShapeDtypeStruct((NC, NP, CW), f32),
                  jax.ShapeDtypeStruct((NC * NP,), f32)),
        mesh=mesh,
        scratch_types=(
            [pltpu.VMEM_SHARED((NP, CW), f32),
             pltpu.VMEM_SHARED((NP,), f32)]
            + [pltpu.VMEM((B,), jnp.int32)] * NB
            + [pltpu.VMEM((B,), jnp.int32)] * NB
            + [pltpu.VMEM((B,), jnp.int32)] * NB
            + [pltpu.VMEM((B,), f32)] * NB
            + [pltpu.VMEM((B,), f32)] * NB
            + [pltpu.VMEM((B,), jnp.int32)] * NB
            + [pltpu.VMEM((B,), f32)] * NB
            + [pltpu.VMEM((B, CW), f32)] * NB
            + [pltpu.VMEM((RPT,), f32)]
            + [pltpu.SemaphoreType.DMA] * (3 * NB)
        ),
    )
    def k(h2, asrc, adst, src, dst, agg_o, den_o, accum_sp, den_sp, *rest):
        src_v = rest[0:NB]
        dst_v = rest[NB:2 * NB]
        sidx_v = rest[2 * NB:3 * NB]
        asg_v = rest[3 * NB:4 * NB]
        adg_v = rest[4 * NB:5 * NB]
        dst_c = rest[5 * NB:6 * NB]
        w_v = rest[6 * NB:7 * NB]
        rows_v = rest[7 * NB:8 * NB]
        zbuf = rows_v[0]
        zden = rest[8 * NB]
        semI = rest[8 * NB + 1:8 * NB + 1 + NB]
        semS = rest[8 * NB + 1 + NB:8 * NB + 1 + 2 * NB]
        semR = rest[8 * NB + 1 + 2 * NB:8 * NB + 1 + 3 * NB]
        c = lax.axis_index("c")
        sid = lax.axis_index("s")

        def zrow(r, carry):
            for j in range(CW // L):
                zbuf[r, pl.ds(j * L, L)] = jnp.zeros((L,), f32)
            return carry
        lax.fori_loop(0, ZR, zrow, 0)
        for j in range(RPT // L):
            zden[pl.ds(j * L, L)] = jnp.zeros((L,), f32)
        for kk in range(RPT // ZR):
            pltpu.sync_copy(zbuf, accum_sp.at[pl.ds(sid * RPT + kk * ZR, ZR)])
        pltpu.sync_copy(zden, den_sp.at[pl.ds(sid * RPT, RPT)])
        plsc.subcore_barrier()

        if split_edges:
            g0 = c * GC
            glim = g0 + GC
            coff = 0
        else:
            g0 = 0
            glim = G
            coff = c * N

        def fire_idx(g, b):
            @pl.when(g < glim)
            def _():
                pltpu.async_copy(src.at[g], src_v[b], semI[b])
                pltpu.async_copy(dst.at[g], dst_v[b], semI[b])

        def wait_idx_fire_gath(g, b):
            @pl.when(g < glim)
            def _():
                pltpu.make_async_copy(src.at[0], src_v[b], semI[b]).wait()
                pltpu.make_async_copy(dst.at[0], dst_v[b], semI[b]).wait()
                pltpu.async_copy(asrc.at[src_v[b]], asg_v[b], semS[b])
                pltpu.async_copy(adst.at[dst_v[b]], adg_v[b], semS[b])
                if split_edges:
                    ridx = src_v[b]
                else:
                    for j in range(B // L):
                        sl = pl.ds(j * L, L)
                        sidx_v[b][sl] = src_v[b][sl] + coff
                    ridx = sidx_v[b]
                pltpu.async_copy(h2.at[ridx], rows_v[b], semR[b])

        def wait_gath_weights(g, b):
            @pl.when(g < glim)
            def _():
                pltpu.make_async_copy(
                    asrc.at[pl.ds(0, B)], asg_v[b], semS[b]).wait()
                pltpu.make_async_copy(
                    adst.at[pl.ds(0, B)], adg_v[b], semS[b]).wait()
                for j in range(B // L):
                    sl = pl.ds(j * L, L)
                    a = asg_v[b][sl] + adg_v[b][sl]
                    e = jnp.where(a >= 0.0, a, 0.2 * a)
                    w_v[b][sl] = jnp.exp(e)
                    dst_c[b][sl] = dst_v[b][sl]
                pltpu.make_async_copy(
                    h2.at[pl.ds(0, B)], rows_v[b], semR[b]).wait()

        def scale_scatter(g, b):
            @pl.when(g < glim)
            def _():
                def srow16(eb, carry2):
                    w16 = w_v[b][pl.ds(eb * L, L)]
                    for r2 in range(L):
                        wb = _bcast_lane(w16, r2)
                        r = eb * L + r2
                        for j2 in range(CW // L):
                            sl2 = pl.ds(j2 * L, L)
                            rows_v[b][r, sl2] = rows_v[b][r, sl2] * wb
                    return carry2
                lax.fori_loop(0, B // L, srow16, 0)
                pltpu.sync_copy(rows_v[b], accum_sp.at[dst_c[b]], add=True)
                pltpu.sync_copy(w_v[b], den_sp.at[dst_c[b]], add=True)

        ga0 = g0 + sid
        fire_idx(ga0, 0)
        wait_idx_fire_gath(ga0, 0)
        fire_idx(ga0 + NS, 1)

        def body(it, carry):
            ga = g0 + sid + NS * 2 * it
            gb = ga + NS
            gc = ga + 2 * NS
            gd = ga + 3 * NS
            wait_idx_fire_gath(gb, 1)
            wait_gath_weights(ga, 0)
            fire_idx(gc, 0)
            scale_scatter(ga, 0)
            wait_idx_fire_gath(gc, 0)
            wait_gath_weights(gb, 1)
            fire_idx(gd, 1)
            scale_scatter(gb, 1)
            return carry
        lax.fori_loop(0, NIT, body, 0)

        plsc.subcore_barrier()

        for kk in range(RPT // ZR):
            r0 = sid * RPT + kk * ZR
            pltpu.sync_copy(accum_sp.at[pl.ds(r0, ZR)], zbuf)
            pltpu.sync_copy(zbuf, agg_o.at[c, pl.ds(r0, ZR)])
        pltpu.sync_copy(den_sp.at[pl.ds(sid * RPT, RPT)], zden)
        pltpu.sync_copy(zden, den_o.at[pl.ds(c * NP + sid * RPT, RPT)])

    return k


def _tc_pre(D, C):
    BN = 400
    f32 = jnp.float32

    def body(x_r, w_r, s_r, d_r, h_r, as_r, ad_r):
        h = jnp.dot(x_r[...], w_r[...], preferred_element_type=f32,
                    precision=lax.Precision.HIGHEST)
        h_r[...] = h
        as_r[...] = jnp.sum(h * s_r[...], axis=1, keepdims=True)
        ad_r[...] = jnp.sum(h * d_r[...], axis=1, keepdims=True)

    return pl.pallas_call(
        body,
        grid=(N // BN,),
        in_specs=[
            pl.BlockSpec((BN, D), lambda i: (i, 0)),
            pl.BlockSpec((D, C), lambda i: (0, 0)),
            pl.BlockSpec((1, C), lambda i: (0, 0)),
            pl.BlockSpec((1, C), lambda i: (0, 0)),
        ],
        out_specs=[
            pl.BlockSpec((BN, C), lambda i: (i, 0)),
            pl.BlockSpec((BN, 1), lambda i: (i, 0)),
            pl.BlockSpec((BN, 1), lambda i: (i, 0)),
        ],
        out_shape=[
            jax.ShapeDtypeStruct((N, C), f32),
            jax.ShapeDtypeStruct((N, 1), f32),
            jax.ShapeDtypeStruct((N, 1), f32),
        ],
    )


def _tc_mid(relu_D, C1, C2o):
    BN = 400
    f32 = jnp.float32

    def body(a0_r, a1_r, d0_r, d1_r, x_r, wr_r, b_r, w2_r, s2_r, t2_r,
             h2_r, as_r, ad_r, hm_r):
        agg = a0_r[0] + a1_r[0]
        den = d0_r[...] + d1_r[...]
        r = jnp.dot(x_r[...], wr_r[...], preferred_element_type=f32,
                    precision=lax.Precision.HIGHEST) + b_r[...]
        hm = jnp.maximum(agg / (den + 1e-16) + r, 0.0)
        hm_r[...] = hm
        h2 = jnp.dot(hm, w2_r[...], preferred_element_type=f32,
                     precision=lax.Precision.HIGHEST)
        as_r[...] = jnp.sum(h2 * s2_r[...], axis=1, keepdims=True)
        ad_r[...] = jnp.sum(h2 * t2_r[...], axis=1, keepdims=True)
        h2_r[0] = h2[:, :CW]
        h2_r[1] = h2[:, CW:]

    return pl.pallas_call(
        body,
        grid=(N // BN,),
        in_specs=[
            pl.BlockSpec((1, BN, CW), lambda i: (0, i, 0)),
            pl.BlockSpec((1, BN, CW), lambda i: (1, i, 0)),
            pl.BlockSpec((BN, 1), lambda i: (i, 0)),
            pl.BlockSpec((BN, 1), lambda i: (i, 0)),
            pl.BlockSpec((BN, relu_D), lambda i: (i, 0)),
            pl.BlockSpec((relu_D, C1), lambda i: (0, 0)),
            pl.BlockSpec((1, C1), lambda i: (0, 0)),
            pl.BlockSpec((C1, C2o), lambda i: (0, 0)),
            pl.BlockSpec((1, C2o), lambda i: (0, 0)),
            pl.BlockSpec((1, C2o), lambda i: (0, 0)),
        ],
        out_specs=[
            pl.BlockSpec((2, BN, CW), lambda i: (0, i, 0)),
            pl.BlockSpec((BN, 1), lambda i: (i, 0)),
            pl.BlockSpec((BN, 1), lambda i: (i, 0)),
            pl.BlockSpec((BN, C1), lambda i: (i, 0)),
        ],
        out_shape=[
            jax.ShapeDtypeStruct((2, N, CW), f32),
            jax.ShapeDtypeStruct((N, 1), f32),
            jax.ShapeDtypeStruct((N, 1), f32),
            jax.ShapeDtypeStruct((N, C1), f32),
        ],
    )


def _tc_fin(D, C):
    BN = 400
    f32 = jnp.float32

    def body(a0_r, a1_r, d_r, x_r, wr_r, b_r, o_r):
        r = jnp.dot(x_r[...], wr_r[...], preferred_element_type=f32,
                    precision=lax.Precision.HIGHEST) + b_r[...]
        dn = d_r[...] + 1e-16
        o_r[...] = jnp.concatenate(
            [a0_r[0] / dn, a1_r[0] / dn], axis=1) + r

    return pl.pallas_call(
        body,
        grid=(N // BN,),
        in_specs=[
            pl.BlockSpec((1, BN, CW), lambda i: (0, i, 0)),
            pl.BlockSpec((1, BN, CW), lambda i: (1, i, 0)),
            pl.BlockSpec((BN, 1), lambda i: (i, 0)),
            pl.BlockSpec((BN, D), lambda i: (i, 0)),
            pl.BlockSpec((D, C), lambda i: (0, 0)),
            pl.BlockSpec((1, C), lambda i: (0, 0)),
        ],
        out_specs=pl.BlockSpec((BN, C), lambda i: (i, 0)),
        out_shape=jax.ShapeDtypeStruct((N, C), f32),
    )


def kernel(x, edge_index, W1, att_src1, att_dst1, W_res1, b1,
           W2, att_src2, att_dst2, W_res2, b2):
    src2 = edge_index[0].astype(jnp.int32).reshape(G, B)
    dst2 = edge_index[1].astype(jnp.int32).reshape(G, B)
    h1, as1, ad1 = _tc_pre(128, 128)(x, W1, att_src1.reshape(1, 128),
                                     att_dst1.reshape(1, 128))
    aggf1, denf1 = _sc_edge(True)(h1, as1.reshape(N), ad1.reshape(N),
                                  src2, dst2)
    h2st, as2, ad2, hmid = _tc_mid(128, 128, 256)(
        aggf1, aggf1, denf1[:N].reshape(N, 1),
        denf1[NP:NP + N].reshape(N, 1),
        x, W_res1, b1.reshape(1, 128), W2, att_src2.reshape(1, 256),
        att_dst2.reshape(1, 256))
    aggf2, denf2 = _sc_edge(False)(h2st.reshape(2 * N, CW),
                                   as2.reshape(N), ad2.reshape(N),
                                   src2, dst2)
    return _tc_fin(128, 256)(aggf2, aggf2, denf2[:N].reshape(N, 1),
                             hmid, W_res2, b2.reshape(1, 256))
